# baseline (speedup 1.0000x reference)
import functools

import jax
import jax.numpy as jnp
from jax import lax
from jax.experimental import pallas as pl
from jax.experimental.pallas import tpu as pltpu

_VMEM_LIMIT_BYTES = 57 * 1024 * 1024
_MB = 256


def _style_loss_kernel(f_ref, tgt_hbm, xout_hbm, loss_ref, acc_ref, tgt_ref,
                       xsem, tsem, *, m, kt, tc, inv_norm, inv_numel):
    cc = pl.program_id(0)
    nb = m // _MB

    @pl.when(cc == 0)
    def _():
        acc_ref[...] = jnp.zeros_like(acc_ref)
        pltpu.make_async_copy(tgt_hbm, tgt_ref, tsem).start()

    xcopy = pltpu.make_async_copy(
        f_ref, xout_hbm.at[:, :, pl.ds(cc * tc, tc), :], xsem)
    xcopy.start()

    v = f_ref[...]
    vb = v.astype(jnp.bfloat16).reshape(m, kt)
    for j in range(nb):
        rj = (j + 1) * _MB
        acc_ref[0:rj, j * _MB:rj] += lax.dot_general(
            vb[0:rj, :], vb[j * _MB:rj, :],
            (((1,), (1,)), ((), ())), preferred_element_type=jnp.float32)

    xcopy.wait()

    @pl.when(cc == pl.num_programs(0) - 1)
    def _():
        pltpu.make_async_copy(tgt_hbm, tgt_ref, tsem).wait()
        total = jnp.float32(0.0)
        for j in range(nb):
            for i in range(j + 1):
                g = acc_ref[i * _MB:(i + 1) * _MB, j * _MB:(j + 1) * _MB]
                t = tgt_ref[i * _MB:(i + 1) * _MB, j * _MB:(j + 1) * _MB]
                diff = g * inv_norm - t
                w = 1.0 if i == j else 2.0
                total += w * jnp.sum(diff * diff)
        loss_ref[0, 0] = total * inv_numel


def kernel(x, target_gram):
    a, b, c, d = x.shape
    m = a * b
    tc = 8
    nc = c // tc
    kt = tc * d
    inv_norm = 1.0 / float(a * b * c * d)
    inv_numel = 1.0 / float(m * m)

    cost = pl.CostEstimate(
        flops=2 * m * m * c * d * 5 // 8,
        transcendentals=0,
        bytes_accessed=2 * a * b * c * d * 4 + m * m * 4)

    x_out, loss = pl.pallas_call(
        functools.partial(_style_loss_kernel, m=m, kt=kt, tc=tc,
                          inv_norm=inv_norm, inv_numel=inv_numel),
        out_shape=[
            jax.ShapeDtypeStruct((a, b, c, d), jnp.float32),
            jax.ShapeDtypeStruct((1, 1), jnp.float32),
        ],
        grid_spec=pltpu.PrefetchScalarGridSpec(
            num_scalar_prefetch=0,
            grid=(nc,),
            in_specs=[
                pl.BlockSpec((a, b, tc, d), lambda cc: (0, 0, cc, 0)),
                pl.BlockSpec(memory_space=pl.ANY),
            ],
            out_specs=[
                pl.BlockSpec(memory_space=pl.ANY),
                pl.BlockSpec((1, 1), lambda cc: (0, 0),
                             memory_space=pltpu.SMEM),
            ],
            scratch_shapes=[
                pltpu.VMEM((m, m), jnp.float32),
                pltpu.VMEM((m, m), jnp.float32),
                pltpu.SemaphoreType.DMA,
                pltpu.SemaphoreType.DMA,
            ],
        ),
        compiler_params=pltpu.CompilerParams(
            dimension_semantics=("arbitrary",),
            vmem_limit_bytes=_VMEM_LIMIT_BYTES),
        cost_estimate=cost,
    )(x, target_gram)

    return x_out, loss[0, 0]


# speedup vs baseline: 1.1182x; 1.1182x over previous
import functools

import jax
import jax.numpy as jnp
from jax import lax
from jax.experimental import pallas as pl
from jax.experimental.pallas import tpu as pltpu

_VMEM_LIMIT_BYTES = 57 * 1024 * 1024
_MB = 256


def _style_loss_kernel(f_ref, tgt_hbm, xout_hbm, loss_ref, acc_ref, tgt_ref,
                       xsem, tsem, *, m, kt, tc, inv_norm, inv_numel):
    cc = pl.program_id(0)
    nb = m // _MB

    @pl.when(cc == 0)
    def _():
        acc_ref[...] = jnp.zeros_like(acc_ref)
        pltpu.make_async_copy(tgt_hbm, tgt_ref, tsem).start()

    xcopy = pltpu.make_async_copy(
        f_ref, xout_hbm.at[:, :, pl.ds(cc * tc, tc), :], xsem)
    xcopy.start()

    v = f_ref[...]
    vb = v.astype(jnp.bfloat16).reshape(m, kt)
    for j in range(nb):
        rj = (j + 1) * _MB
        acc_ref[0:rj, j * _MB:rj] += lax.dot_general(
            vb[0:rj, :], vb[j * _MB:rj, :],
            (((1,), (1,)), ((), ())), preferred_element_type=jnp.float32)

    xcopy.wait()

    @pl.when(cc == pl.num_programs(0) - 1)
    def _():
        pltpu.make_async_copy(tgt_hbm, tgt_ref, tsem).wait()
        total = jnp.float32(0.0)
        for j in range(nb):
            for i in range(j + 1):
                g = acc_ref[i * _MB:(i + 1) * _MB, j * _MB:(j + 1) * _MB]
                t = tgt_ref[i * _MB:(i + 1) * _MB, j * _MB:(j + 1) * _MB]
                diff = g * inv_norm - t
                w = 1.0 if i == j else 2.0
                total += w * jnp.sum(diff * diff)
        loss_ref[0, 0] = total * inv_numel


def kernel(x, target_gram):
    a, b, c, d = x.shape
    m = a * b
    tc = 16
    nc = c // tc
    kt = tc * d
    inv_norm = 1.0 / float(a * b * c * d)
    inv_numel = 1.0 / float(m * m)

    cost = pl.CostEstimate(
        flops=2 * m * m * c * d * 5 // 8,
        transcendentals=0,
        bytes_accessed=2 * a * b * c * d * 4 + m * m * 4)

    x_out, loss = pl.pallas_call(
        functools.partial(_style_loss_kernel, m=m, kt=kt, tc=tc,
                          inv_norm=inv_norm, inv_numel=inv_numel),
        out_shape=[
            jax.ShapeDtypeStruct((a, b, c, d), jnp.float32),
            jax.ShapeDtypeStruct((1, 1), jnp.float32),
        ],
        grid_spec=pltpu.PrefetchScalarGridSpec(
            num_scalar_prefetch=0,
            grid=(nc,),
            in_specs=[
                pl.BlockSpec((a, b, tc, d), lambda cc: (0, 0, cc, 0)),
                pl.BlockSpec(memory_space=pl.ANY),
            ],
            out_specs=[
                pl.BlockSpec(memory_space=pl.ANY),
                pl.BlockSpec((1, 1), lambda cc: (0, 0),
                             memory_space=pltpu.SMEM),
            ],
            scratch_shapes=[
                pltpu.VMEM((m, m), jnp.float32),
                pltpu.VMEM((m, m), jnp.float32),
                pltpu.SemaphoreType.DMA,
                pltpu.SemaphoreType.DMA,
            ],
        ),
        compiler_params=pltpu.CompilerParams(
            dimension_semantics=("arbitrary",),
            vmem_limit_bytes=_VMEM_LIMIT_BYTES),
        cost_estimate=cost,
    )(x, target_gram)

    return x_out, loss[0, 0]


# speedup vs baseline: 1.1340x; 1.0141x over previous
import functools

import jax
import jax.numpy as jnp
from jax import lax
from jax.experimental import pallas as pl
from jax.experimental.pallas import tpu as pltpu

_VMEM_LIMIT_BYTES = 57 * 1024 * 1024
_MB = 256


def _style_loss_kernel(f_ref, tgt_hbm, xout_hbm, loss_ref, acc_ref, tgt_ref,
                       xsem, tsem, *, m, kt, tc, inv_norm, inv_numel):
    cc = pl.program_id(0)
    nb = m // _MB

    @pl.when(cc == 0)
    def _():
        acc_ref[...] = jnp.zeros_like(acc_ref)
        pltpu.make_async_copy(tgt_hbm, tgt_ref, tsem).start()

    xcopy = pltpu.make_async_copy(
        f_ref, xout_hbm.at[:, :, pl.ds(cc * tc, tc), :], xsem)
    xcopy.start()

    v = f_ref[...]
    vb = v.astype(jnp.bfloat16).reshape(m, kt)
    for j in range(nb):
        rj = (j + 1) * _MB
        acc_ref[0:rj, j * _MB:rj] += lax.dot_general(
            vb[0:rj, :], vb[j * _MB:rj, :],
            (((1,), (1,)), ((), ())), preferred_element_type=jnp.float32)

    xcopy.wait()

    @pl.when(cc == pl.num_programs(0) - 1)
    def _():
        pltpu.make_async_copy(tgt_hbm, tgt_ref, tsem).wait()
        total = jnp.float32(0.0)
        for j in range(nb):
            for i in range(j + 1):
                g = acc_ref[i * _MB:(i + 1) * _MB, j * _MB:(j + 1) * _MB]
                t = tgt_ref[i * _MB:(i + 1) * _MB, j * _MB:(j + 1) * _MB]
                diff = g * inv_norm - t
                w = 1.0 if i == j else 2.0
                total += w * jnp.sum(diff * diff)
        loss_ref[0, 0] = total * inv_numel


def kernel(x, target_gram):
    a, b, c, d = x.shape
    m = a * b
    tc = 32
    nc = c // tc
    kt = tc * d
    inv_norm = 1.0 / float(a * b * c * d)
    inv_numel = 1.0 / float(m * m)

    cost = pl.CostEstimate(
        flops=2 * m * m * c * d * 5 // 8,
        transcendentals=0,
        bytes_accessed=2 * a * b * c * d * 4 + m * m * 4)

    x_out, loss = pl.pallas_call(
        functools.partial(_style_loss_kernel, m=m, kt=kt, tc=tc,
                          inv_norm=inv_norm, inv_numel=inv_numel),
        out_shape=[
            jax.ShapeDtypeStruct((a, b, c, d), jnp.float32),
            jax.ShapeDtypeStruct((1, 1), jnp.float32),
        ],
        grid_spec=pltpu.PrefetchScalarGridSpec(
            num_scalar_prefetch=0,
            grid=(nc,),
            in_specs=[
                pl.BlockSpec((a, b, tc, d), lambda cc: (0, 0, cc, 0)),
                pl.BlockSpec(memory_space=pl.ANY),
            ],
            out_specs=[
                pl.BlockSpec(memory_space=pl.ANY),
                pl.BlockSpec((1, 1), lambda cc: (0, 0),
                             memory_space=pltpu.SMEM),
            ],
            scratch_shapes=[
                pltpu.VMEM((m, m), jnp.float32),
                pltpu.VMEM((m, m), jnp.float32),
                pltpu.SemaphoreType.DMA,
                pltpu.SemaphoreType.DMA,
            ],
        ),
        compiler_params=pltpu.CompilerParams(
            dimension_semantics=("arbitrary",),
            vmem_limit_bytes=_VMEM_LIMIT_BYTES),
        cost_estimate=cost,
    )(x, target_gram)

    return x_out, loss[0, 0]
